# baseline (speedup 1.0000x reference)
import functools

import jax
import jax.numpy as jnp
from jax.experimental import pallas as pl
from jax.experimental.pallas import tpu as pltpu

NUM_CODES = 1024
D = 64
BLK = 512


def _bdot(a, b, dims=(((1,), (1,)), ((), ()))):
    return jax.lax.dot_general(a.astype(jnp.bfloat16), b.astype(jnp.bfloat16),
                               dims, preferred_element_type=jnp.float32)


def _body(x_ref, cb_ref, we1, be1, we2, be2, wd1, bd1, wd2, bd2,
          out_ref, tbl_ref, csq_ref):
    pid = pl.program_id(0)

    @pl.when(pid == 0)
    def _init():
        cb = cb_ref[...]
        h = jnp.maximum(
            jnp.dot(cb.astype(jnp.bfloat16), wd1[...].astype(jnp.bfloat16),
                    preferred_element_type=jnp.float32) + bd1[...], 0.0)
        tbl_ref[...] = (jnp.dot(h.astype(jnp.bfloat16),
                                wd2[...].astype(jnp.bfloat16),
                                preferred_element_type=jnp.float32) + bd2[...])
        csq_ref[...] = jnp.sum(cb * cb, axis=1, keepdims=True)

    h1 = jnp.maximum(
        _bdot(x_ref[...], we1[...], (((1,), (0,)), ((), ()))) + be1[...], 0.0)
    flat = _bdot(h1, we2[...], (((1,), (0,)), ((), ()))) + be2[...]

    fsq_row = jnp.sum(flat * flat, axis=1, keepdims=True).T
    cross_t = _bdot(2.0 * cb_ref[...], flat)
    dist = (fsq_row - cross_t) + csq_ref[...]

    val = tbl_ref[...]
    w = NUM_CODES // 2
    cond = dist[:w, :] <= dist[w:, :]
    d = jnp.where(cond, dist[:w, :], dist[w:, :])
    v = jnp.where(cond, val[:w, :], val[w:, :])
    w //= 2
    while w >= 1:
        cond = d[:w, :] <= d[w:, :]
        d = jnp.where(cond, d[:w, :], d[w:, :])
        v = jnp.where(cond, v[:w, :], v[w:, :])
        w //= 2
    out_ref[...] = v[None]


@functools.partial(jax.jit, static_argnames=("interpret",))
def _run(x, codebook, W_enc1, b_enc1, W_enc2, b_enc2,
         W_dec1, b_dec1, W_dec2, b_dec2, interpret=False):
    T, B, _ = x.shape
    N = T * B
    xf = x.reshape(N, 2)
    grid = (N // BLK,)
    out = pl.pallas_call(
        _body,
        grid=grid,
        in_specs=[
            pl.BlockSpec((BLK, 2), lambda i: (i, 0)),
            pl.BlockSpec((NUM_CODES, D), lambda i: (0, 0)),
            pl.BlockSpec((2, D), lambda i: (0, 0)),
            pl.BlockSpec((1, D), lambda i: (0, 0)),
            pl.BlockSpec((D, D), lambda i: (0, 0)),
            pl.BlockSpec((1, D), lambda i: (0, 0)),
            pl.BlockSpec((D, D), lambda i: (0, 0)),
            pl.BlockSpec((1, D), lambda i: (0, 0)),
            pl.BlockSpec((D, 1), lambda i: (0, 0)),
            pl.BlockSpec((1, 1), lambda i: (0, 0)),
        ],
        out_specs=pl.BlockSpec((1, 1, BLK), lambda i: (i, 0, 0)),
        out_shape=jax.ShapeDtypeStruct((N // BLK, 1, BLK), jnp.float32),
        scratch_shapes=[
            pltpu.VMEM((NUM_CODES, 1), jnp.float32),
            pltpu.VMEM((NUM_CODES, 1), jnp.float32),
        ],
        interpret=interpret,
    )(xf, codebook,
      W_enc1, b_enc1.reshape(1, D),
      W_enc2, b_enc2.reshape(1, D),
      W_dec1, b_dec1.reshape(1, D),
      W_dec2, b_dec2.reshape(1, 1))
    return out.reshape(T, B, 1)


def kernel(x, codebook, W_enc1, b_enc1, W_enc2, b_enc2,
           W_dec1, b_dec1, W_dec2, b_dec2):
    return _run(x, codebook, W_enc1, b_enc1, W_enc2, b_enc2,
                W_dec1, b_dec1, W_dec2, b_dec2)


# speedup vs baseline: 1.5076x; 1.5076x over previous
import functools

import jax
import jax.numpy as jnp
from jax.experimental import pallas as pl
from jax.experimental.pallas import tpu as pltpu

NUM_CODES = 1024
D = 64
BLK = 1024


def _bdot(a, b, dims):
    return jax.lax.dot_general(a.astype(jnp.bfloat16), b.astype(jnp.bfloat16),
                               dims, preferred_element_type=jnp.float32)


def _body(xt_ref, cb_ref, we1, be1, we2, be2, wd1, bd1, wd2, bd2,
          out_ref, tbl_ref, csq_ref):
    pid = pl.program_id(0)

    @pl.when(pid == 0)
    def _init():
        cb = cb_ref[...]
        h = jnp.maximum(
            _bdot(cb, wd1[...], (((1,), (0,)), ((), ()))) + bd1[...], 0.0)
        tbl_ref[...] = _bdot(h, wd2[...], (((1,), (0,)), ((), ()))) + bd2[...]
        csq_ref[...] = jnp.sum(cb * cb, axis=1, keepdims=True)

    h1t = jnp.maximum(
        _bdot(we1[...], xt_ref[...], (((0,), (0,)), ((), ()))) + be1[...], 0.0)
    flat_t = _bdot(we2[...], h1t, (((0,), (0,)), ((), ()))) + be2[...]

    fsq_row = jnp.sum(flat_t * flat_t, axis=0, keepdims=True)
    cross_t = _bdot(2.0 * cb_ref[...], flat_t, (((1,), (0,)), ((), ())))
    dist = (fsq_row - cross_t) + csq_ref[...]

    val = tbl_ref[...]
    w = NUM_CODES // 2
    cond = dist[:w, :] <= dist[w:, :]
    d = jnp.where(cond, dist[:w, :], dist[w:, :])
    v = jnp.where(cond, val[:w, :], val[w:, :])
    w //= 2
    while w >= 1:
        cond = d[:w, :] <= d[w:, :]
        d = jnp.where(cond, d[:w, :], d[w:, :])
        v = jnp.where(cond, v[:w, :], v[w:, :])
        w //= 2
    out_ref[...] = v[None]


@functools.partial(jax.jit, static_argnames=("interpret",))
def _run(x, codebook, W_enc1, b_enc1, W_enc2, b_enc2,
         W_dec1, b_dec1, W_dec2, b_dec2, interpret=False):
    T, B, _ = x.shape
    N = T * B
    xt = x.reshape(N, 2).T
    grid = (N // BLK,)
    out = pl.pallas_call(
        _body,
        grid=grid,
        in_specs=[
            pl.BlockSpec((2, BLK), lambda i: (0, i)),
            pl.BlockSpec((NUM_CODES, D), lambda i: (0, 0)),
            pl.BlockSpec((2, D), lambda i: (0, 0)),
            pl.BlockSpec((D, 1), lambda i: (0, 0)),
            pl.BlockSpec((D, D), lambda i: (0, 0)),
            pl.BlockSpec((D, 1), lambda i: (0, 0)),
            pl.BlockSpec((D, D), lambda i: (0, 0)),
            pl.BlockSpec((1, D), lambda i: (0, 0)),
            pl.BlockSpec((D, 1), lambda i: (0, 0)),
            pl.BlockSpec((1, 1), lambda i: (0, 0)),
        ],
        out_specs=pl.BlockSpec((1, 1, BLK), lambda i: (i, 0, 0)),
        out_shape=jax.ShapeDtypeStruct((N // BLK, 1, BLK), jnp.float32),
        scratch_shapes=[
            pltpu.VMEM((NUM_CODES, 1), jnp.float32),
            pltpu.VMEM((NUM_CODES, 1), jnp.float32),
        ],
        interpret=interpret,
    )(xt, codebook,
      W_enc1, b_enc1.reshape(D, 1),
      W_enc2, b_enc2.reshape(D, 1),
      W_dec1, b_dec1.reshape(1, D),
      W_dec2, b_dec2.reshape(1, 1))
    return out.reshape(T, B, 1)


def kernel(x, codebook, W_enc1, b_enc1, W_enc2, b_enc2,
           W_dec1, b_dec1, W_dec2, b_dec2):
    return _run(x, codebook, W_enc1, b_enc1, W_enc2, b_enc2,
                W_dec1, b_dec1, W_dec2, b_dec2)
